# baseline (speedup 1.0000x reference)
import functools

import jax
import jax.numpy as jnp
from jax import lax
from jax.experimental import pallas as pl
from jax.experimental.pallas import tpu as pltpu
from jax.experimental.pallas import tpu_sc as plsc

ALPHA = 0.1
HALF = 32
N_U0 = 25000
N_I0 = 25000
N_INTRA = 50000
N_SEP = 100000
B = 4096

NC = 2
NS = 16

NP_INTRA = 51200
NP_SEP = 100352
NBT_INTRA = 104
NBT_SEP = 200
E_INTRA_P = NBT_INTRA * NS * 512
E_SEP_P = NBT_SEP * NS * 512

_MESH = plsc.VectorSubcoreMesh(
    core_axis_name="c", subcore_axis_name="s", num_cores=NC, num_subcores=NS
)
_SC_PARAMS = pltpu.CompilerParams(
    use_tc_tiling_on_sc=False, needs_layout_passes=False
)


def _pad_edges(src, dst, n, e_pad):
    e = src.shape[0]
    fill = n + (jnp.arange(e_pad - e, dtype=jnp.int32) % 128)
    s1 = jnp.concatenate([src.astype(jnp.int32), fill]).reshape(-1, 4, 128)
    d1 = jnp.concatenate([dst.astype(jnp.int32), fill]).reshape(-1, 4, 128)
    return jnp.stack([s1, d1], axis=1)


def _deg_body(e4i, e4s0, e4s1, zi,
              do_i, dn_i, do_s0, dn_s0, do_s1, dn_s1,
              acc_i, acc_s0, acc_s1,
              eb0, eb1, eb2, eb3, ones_v,
              es0, es1, es2, es3, ss0, ss1):
    c = lax.axis_index("c")
    s = lax.axis_index("s")
    ebuf = (eb0, eb1, eb2, eb3)
    esem = (es0, es1, es2, es3)
    ssem = (ss0, ss1)
    for j in range(8):
        ones_v[pl.ds(j * 16, 16)] = jnp.ones((16,), jnp.int32)
    rpt_i = NP_INTRA // NS
    rpt_s = NP_SEP // NS
    pltpu.sync_copy(zi.at[pl.ds(0, rpt_i)], acc_i.at[pl.ds(s * rpt_i, rpt_i)])
    pltpu.sync_copy(zi.at[pl.ds(0, rpt_s)], acc_s0.at[pl.ds(s * rpt_s, rpt_s)])
    pltpu.sync_copy(zi.at[pl.ds(0, rpt_s)], acc_s1.at[pl.ds(s * rpt_s, rpt_s)])
    plsc.subcore_barrier()

    def run_graph(e4, acc, nbt):
        def emit(kt, e, p, drain):
            if drain:
                epv = (e + 2) % 4
                for j in range(4):
                    pltpu.make_async_copy(ones_v, acc.at[ebuf[epv].at[j]], ssem[p]).wait()
            nxt = kt + 2
            if isinstance(kt, int):
                nxt = nxt if nxt < nbt else nxt - nbt
            else:
                nxt = jnp.where(nxt >= nbt, nxt - nbt, nxt)
            pltpu.async_copy(e4.at[nxt * NS + s, c], ebuf[(e + 2) % 4], esem[(e + 2) % 4])
            pltpu.make_async_copy(e4.at[kt * NS + s, c], ebuf[e], esem[e]).wait()
            for j in range(4):
                pltpu.async_copy(ones_v, acc.at[ebuf[e].at[j]], ssem[p], add=True)

        pltpu.async_copy(e4.at[s, c], eb0, es0)
        pltpu.async_copy(e4.at[NS + s, c], eb1, es1)
        emit(0, 0, 0, False)
        emit(1, 1, 1, False)
        emit(2, 2, 0, True)
        emit(3, 3, 1, True)

        def outer(o4, carry):
            for j in range(4):
                emit(o4 * 4 + j, j, j % 2, True)
            return carry

        lax.fori_loop(1, nbt // 4, outer, 0)
        pltpu.make_async_copy(e4.at[s, c], eb0, es0).wait()
        pltpu.make_async_copy(e4.at[NS + s, c], eb1, es1).wait()
        for j in range(4):
            pltpu.make_async_copy(ones_v, acc.at[ebuf[2].at[j]], ssem[0]).wait()
        for j in range(4):
            pltpu.make_async_copy(ones_v, acc.at[ebuf[3].at[j]], ssem[1]).wait()

    run_graph(e4i, acc_i, NBT_INTRA)
    run_graph(e4s0, acc_s0, NBT_SEP)
    run_graph(e4s1, acc_s1, NBT_SEP)
    plsc.subcore_barrier()

    @pl.when(c == 0)
    def _():
        pltpu.sync_copy(acc_i.at[pl.ds(s * rpt_i, rpt_i)], do_i.at[pl.ds(s * rpt_i, rpt_i)])
        pltpu.sync_copy(acc_s0.at[pl.ds(s * rpt_s, rpt_s)], do_s0.at[pl.ds(s * rpt_s, rpt_s)])
        pltpu.sync_copy(acc_s1.at[pl.ds(s * rpt_s, rpt_s)], do_s1.at[pl.ds(s * rpt_s, rpt_s)])

    @pl.when(c == 1)
    def _():
        pltpu.sync_copy(acc_i.at[pl.ds(s * rpt_i, rpt_i)], dn_i.at[pl.ds(s * rpt_i, rpt_i)])
        pltpu.sync_copy(acc_s0.at[pl.ds(s * rpt_s, rpt_s)], dn_s0.at[pl.ds(s * rpt_s, rpt_s)])
        pltpu.sync_copy(acc_s1.at[pl.ds(s * rpt_s, rpt_s)], dn_s1.at[pl.ds(s * rpt_s, rpt_s)])


_deg_kernel = functools.partial(
    pl.kernel,
    _deg_body,
    out_type=tuple(
        jax.ShapeDtypeStruct((np_,), jnp.int32)
        for np_ in (NP_INTRA, NP_INTRA, NP_SEP, NP_SEP, NP_SEP, NP_SEP)
    ),
    mesh=_MESH,
    compiler_params=_SC_PARAMS,
    scratch_types=[
        pltpu.VMEM_SHARED((NP_INTRA,), jnp.int32),
        pltpu.VMEM_SHARED((NP_SEP,), jnp.int32),
        pltpu.VMEM_SHARED((NP_SEP,), jnp.int32),
        pltpu.VMEM((4, 128), jnp.int32),
        pltpu.VMEM((4, 128), jnp.int32),
        pltpu.VMEM((4, 128), jnp.int32),
        pltpu.VMEM((4, 128), jnp.int32),
        pltpu.VMEM((128,), jnp.int32),
        pltpu.SemaphoreType.DMA,
        pltpu.SemaphoreType.DMA,
        pltpu.SemaphoreType.DMA,
        pltpu.SemaphoreType.DMA,
        pltpu.SemaphoreType.DMA,
        pltpu.SemaphoreType.DMA,
    ],
)()


def _make_ep(np_, nbt):
    rpt = np_ // NS

    def body(e4, table, init, av, out, acc,
             eb0, eb1, eb2, eb3, r0, r1, fbuf, abuf,
             es0, es1, es2, es3, gs0, gs1, ss0, ss1):
        c = lax.axis_index("c")
        s = lax.axis_index("s")
        ebuf = (eb0, eb1, eb2, eb3)
        rowsb = (r0, r1)
        esem = (es0, es1, es2, es3)
        gsem = (gs0, gs1)
        ssem = (ss0, ss1)
        tm = table.at[c]
        pltpu.sync_copy(init.at[c, pl.ds(s * rpt, rpt)], acc.at[pl.ds(s * rpt, rpt)])
        pltpu.sync_copy(av.at[pl.ds(s * rpt, rpt)], abuf)
        plsc.subcore_barrier()

        def gathers(e, p):
            for j in range(4):
                pltpu.async_copy(tm.at[ebuf[e].at[0, j]], rowsb[p].at[j], gsem[p])

        def scatters_prev(e, p):
            q = 1 - p
            epv = (e + 3) % 4
            for j in range(4):
                pltpu.make_async_copy(tm.at[ebuf[epv].at[0, j]], rowsb[q].at[j], gsem[q]).wait()
            for j in range(4):
                pltpu.async_copy(rowsb[q].at[j], acc.at[ebuf[epv].at[1, j]], ssem[q], add=True)

        def emit(kt, e, p, first, drain):
            if drain:
                epv = (e + 2) % 4
                for j in range(4):
                    pltpu.make_async_copy(rowsb[p].at[j], acc.at[ebuf[epv].at[1, j]], ssem[p]).wait()
            nxt = kt + 2
            if isinstance(kt, int):
                nxt = nxt if nxt < nbt else nxt - nbt
            else:
                nxt = jnp.where(nxt >= nbt, nxt - nbt, nxt)
            pltpu.async_copy(e4.at[nxt * NS + s], ebuf[(e + 2) % 4], esem[(e + 2) % 4])
            pltpu.make_async_copy(e4.at[kt * NS + s], ebuf[e], esem[e]).wait()
            gathers(e, p)
            if not first:
                scatters_prev(e, p)

        pltpu.async_copy(e4.at[s], eb0, es0)
        pltpu.async_copy(e4.at[NS + s], eb1, es1)
        emit(0, 0, 0, True, False)
        emit(1, 1, 1, False, False)
        emit(2, 2, 0, False, True)
        emit(3, 3, 1, False, True)

        def outer(o4, carry):
            for j in range(4):
                emit(o4 * 4 + j, j, j % 2, False, True)
            return carry

        lax.fori_loop(1, nbt // 4, outer, 0)
        for j in range(4):
            pltpu.make_async_copy(tm.at[ebuf[3].at[0, j]], rowsb[1].at[j], gsem[1]).wait()
        for j in range(4):
            pltpu.async_copy(rowsb[1].at[j], acc.at[ebuf[3].at[1, j]], ssem[1], add=True)
        for j in range(4):
            pltpu.make_async_copy(rowsb[0].at[j], acc.at[ebuf[2].at[1, j]], ssem[0]).wait()
        for j in range(4):
            pltpu.make_async_copy(rowsb[1].at[j], acc.at[ebuf[3].at[1, j]], ssem[1]).wait()
        pltpu.make_async_copy(e4.at[s], eb0, es0).wait()
        pltpu.make_async_copy(e4.at[NS + s], eb1, es1).wait()
        plsc.subcore_barrier()

        def scale_chunk(t, carry):
            r0_ = s * rpt + t * 128
            pltpu.sync_copy(acc.at[pl.ds(r0_, 128)], fbuf)
            base = t * 128 + jnp.zeros((16,), jnp.int32)
            for r in range(128):
                arow = plsc.load_gather(abuf, [base + r])
                fbuf[r] = fbuf[r] * arow
            pltpu.sync_copy(fbuf, out.at[c, pl.ds(r0_, 128)])
            return carry

        lax.fori_loop(0, rpt // 128, scale_chunk, 0)

    return pl.kernel(
        body,
        out_type=jax.ShapeDtypeStruct((NC, np_, 16), jnp.float32),
        mesh=_MESH,
        compiler_params=_SC_PARAMS,
        scratch_types=[
            pltpu.VMEM_SHARED((np_, 16), jnp.float32),
            pltpu.VMEM((2, 4, 128), jnp.int32),
            pltpu.VMEM((2, 4, 128), jnp.int32),
            pltpu.VMEM((2, 4, 128), jnp.int32),
            pltpu.VMEM((2, 4, 128), jnp.int32),
            pltpu.VMEM((4, 128, 16), jnp.float32),
            pltpu.VMEM((4, 128, 16), jnp.float32),
            pltpu.VMEM((128, 16), jnp.float32),
            pltpu.VMEM((rpt,), jnp.float32),
            pltpu.SemaphoreType.DMA,
            pltpu.SemaphoreType.DMA,
            pltpu.SemaphoreType.DMA,
            pltpu.SemaphoreType.DMA,
            pltpu.SemaphoreType.DMA,
            pltpu.SemaphoreType.DMA,
            pltpu.SemaphoreType.DMA,
            pltpu.SemaphoreType.DMA,
        ],
    )


_ep_intra = _make_ep(NP_INTRA, NBT_INTRA)
_ep_sep = _make_ep(NP_SEP, NBT_SEP)


def _fin_body(u2d, i2d, utab, itab, uout, iout, idxb, rows, sem):
    c = lax.axis_index("c")
    s = lax.axis_index("s")
    wid = s * NC + c
    pltpu.sync_copy(u2d.at[wid], idxb)
    pltpu.async_copy(utab.at[idxb], rows, sem).wait()
    pltpu.sync_copy(rows, uout.at[pl.ds(wid * 128, 128)])
    pltpu.sync_copy(i2d.at[wid], idxb)
    pltpu.async_copy(itab.at[idxb], rows, sem).wait()
    pltpu.sync_copy(rows, iout.at[pl.ds(wid * 128, 128)])


_fin_gather = functools.partial(
    pl.kernel,
    _fin_body,
    out_type=(
        jax.ShapeDtypeStruct((B, 64), jnp.float32),
        jax.ShapeDtypeStruct((B, 64), jnp.float32),
    ),
    mesh=_MESH,
    compiler_params=_SC_PARAMS,
    scratch_types=[
        pltpu.VMEM((128,), jnp.int32),
        pltpu.VMEM((128, 64), jnp.float32),
        pltpu.SemaphoreType.DMA,
    ],
)()


_BN = 2000


def _prep_tc_body(do_ref, dn_ref, x_ref, g1_ref, init_ref, a1_ref, a2_ref):
    isdo = lax.rsqrt(jnp.maximum(do_ref[...], 1).astype(jnp.float32))
    isdi = lax.rsqrt(jnp.maximum(dn_ref[...], 1).astype(jnp.float32))
    x = x_ref[...]
    g1_ref[0] = x[:, :16] * isdo
    g1_ref[1] = x[:, 16:] * isdo
    w = (1.0 - ALPHA) * isdi
    iscale = ALPHA / w
    init_ref[0] = x[:, :16] * iscale
    init_ref[1] = x[:, 16:] * iscale
    a1_ref[...] = w * isdo
    a2_ref[...] = w


def _prep_tc(deg_out, deg_in, x, np_):
    nb = x.shape[0] // _BN
    row = pl.BlockSpec((_BN, 1), lambda i: (i, 0))
    tab = pl.BlockSpec((NC, _BN, 16), lambda i: (0, i, 0))
    return pl.pallas_call(
        _prep_tc_body,
        grid=(nb,),
        in_specs=[row, row, pl.BlockSpec((_BN, HALF), lambda i: (i, 0))],
        out_specs=[tab, tab, row, row],
        out_shape=[
            jax.ShapeDtypeStruct((NC, np_, 16), jnp.float32),
            jax.ShapeDtypeStruct((NC, np_, 16), jnp.float32),
            jax.ShapeDtypeStruct((np_, 1), jnp.float32),
            jax.ShapeDtypeStruct((np_, 1), jnp.float32),
        ],
    )(deg_out.reshape(np_, 1), deg_in.reshape(np_, 1), x)


_BF = 1000


def _final_tc_body(h0u_ref, h0i_ref, h1u_ref, h1i_ref, hiu_ref, hii_ref,
                   u_ref, i_ref):
    def cat(a_ref):
        return jnp.concatenate([a_ref[0], a_ref[1]], axis=1)

    u_ref[...] = jnp.concatenate(
        [0.5 * (cat(h0u_ref) + cat(h1u_ref)), cat(hiu_ref)], axis=1)
    i_ref[...] = jnp.concatenate(
        [0.5 * (cat(h0i_ref) + cat(h1i_ref)), cat(hii_ref)], axis=1)


def _final_tc(h2_s0, h2_s1, h2_in):
    nb = N_U0 // _BF
    su = 50000 // _BF
    iu = 25000 // _BF
    a_u = pl.BlockSpec((NC, _BF, 16), lambda i: (0, i, 0))
    a_si = pl.BlockSpec((NC, _BF, 16), lambda i: (0, i + su, 0))
    a_ii = pl.BlockSpec((NC, _BF, 16), lambda i: (0, i + iu, 0))
    out = pl.BlockSpec((_BF, 64), lambda i: (i, 0))
    return pl.pallas_call(
        _final_tc_body,
        grid=(nb,),
        in_specs=[a_u, a_si, a_u, a_si, a_u, a_ii],
        out_specs=[out, out],
        out_shape=[
            jax.ShapeDtypeStruct((N_U0, 64), jnp.float32),
            jax.ShapeDtypeStruct((N_I0, 64), jnp.float32),
        ],
    )(h2_s0, h2_s0, h2_s1, h2_s1, h2_in, h2_in)


def _dot_tc_body(u_ref, i_ref, g_ref):
    g_ref[...] = jnp.sum(u_ref[...] * i_ref[...], axis=1)


def _dot_tc(u_rows, i_rows):
    return pl.pallas_call(
        _dot_tc_body,
        out_shape=jax.ShapeDtypeStruct((B,), jnp.float32),
    )(u_rows, i_rows)


def kernel(users, items, edge_index_intra, edge_index_sep0, edge_index_sep1,
           emb_user_d0, emb_item_d0, aggr_user, aggr_item):
    x_intra = jnp.concatenate([emb_user_d0, emb_item_d0], axis=0)
    x_sep = jnp.concatenate([aggr_user, aggr_item], axis=0)

    e4_i = _pad_edges(edge_index_intra[0], edge_index_intra[1], N_INTRA, E_INTRA_P)
    e4_s0 = _pad_edges(edge_index_sep0[0], edge_index_sep0[1], N_SEP, E_SEP_P)
    e4_s1 = _pad_edges(edge_index_sep1[0], edge_index_sep1[1], N_SEP, E_SEP_P)

    zi = jnp.zeros((NP_SEP,), jnp.int32)

    do_i, dn_i, do_s0, dn_s0, do_s1, dn_s1 = _deg_kernel(e4_i, e4_s0, e4_s1, zi)

    g1_i, init_i, a1_i, a2_i = _prep_tc(do_i, dn_i, x_intra, NP_INTRA)
    g1_s0, init_s0, a1_s0, a2_s0 = _prep_tc(do_s0, dn_s0, x_sep, NP_SEP)
    g1_s1, init_s1, a1_s1, a2_s1 = _prep_tc(do_s1, dn_s1, x_sep, NP_SEP)

    g2_i = _ep_intra(e4_i, g1_i, init_i, a1_i.reshape(NP_INTRA))
    g2_s0 = _ep_sep(e4_s0, g1_s0, init_s0, a1_s0.reshape(NP_SEP))
    g2_s1 = _ep_sep(e4_s1, g1_s1, init_s1, a1_s1.reshape(NP_SEP))

    h2_i = _ep_intra(e4_i, g2_i, init_i, a2_i.reshape(NP_INTRA))
    h2_s0 = _ep_sep(e4_s0, g2_s0, init_s0, a2_s0.reshape(NP_SEP))
    h2_s1 = _ep_sep(e4_s1, g2_s1, init_s1, a2_s1.reshape(NP_SEP))

    u_tab, i_tab = _final_tc(h2_s0, h2_s1, h2_i)

    u2d = users.astype(jnp.int32).reshape(B // 128, 128)
    i2d = items.astype(jnp.int32).reshape(B // 128, 128)
    u_rows, i_rows = _fin_gather(u2d, i2d, u_tab, i_tab)
    return _dot_tc(u_rows, i_rows)


# speedup vs baseline: 1.0803x; 1.0803x over previous
import functools

import jax
import jax.numpy as jnp
from jax import lax
from jax.experimental import pallas as pl
from jax.experimental.pallas import tpu as pltpu
from jax.experimental.pallas import tpu_sc as plsc

ALPHA = 0.1
HALF = 32
N_U0 = 25000
N_I0 = 25000
N_INTRA = 50000
N_SEP = 100000
B = 4096

NC = 2
NS = 16

NP_INTRA = 51200
NP_SEP = 100352
NBT_INTRA = 104
NBT_SEP = 200
E_INTRA_P = NBT_INTRA * NS * 512
E_SEP_P = NBT_SEP * NS * 512

_MESH = plsc.VectorSubcoreMesh(
    core_axis_name="c", subcore_axis_name="s", num_cores=NC, num_subcores=NS
)
_SC_PARAMS = pltpu.CompilerParams(
    use_tc_tiling_on_sc=False, needs_layout_passes=False
)


def _pad_edges(src, dst, n, e_pad):
    e = src.shape[0]
    fill = n + (jnp.arange(e_pad - e, dtype=jnp.int32) % 128)
    s1 = jnp.concatenate([src.astype(jnp.int32), fill]).reshape(-1, 4, 128)
    d1 = jnp.concatenate([dst.astype(jnp.int32), fill]).reshape(-1, 4, 128)
    return jnp.stack([s1, d1], axis=1)


def _make_deg(np_, nbt):
    rpt = np_ // NS

    def body(e4, zi, do_, dn_, acc,
             eb0, eb1, eb2, eb3, ones_v,
             es0, es1, es2, es3, ss0, ss1):
        c = lax.axis_index("c")
        s = lax.axis_index("s")
        ebuf = (eb0, eb1, eb2, eb3)
        esem = (es0, es1, es2, es3)
        ssem = (ss0, ss1)
        for j in range(8):
            ones_v[pl.ds(j * 16, 16)] = jnp.ones((16,), jnp.int32)
        pltpu.sync_copy(zi.at[pl.ds(0, rpt)], acc.at[pl.ds(s * rpt, rpt)])
        plsc.subcore_barrier()

        def emit(kt, e, p, drain):
            if drain:
                epv = (e + 2) % 4
                for j in range(4):
                    pltpu.make_async_copy(ones_v, acc.at[ebuf[epv].at[j]], ssem[p]).wait()
            nxt = kt + 2
            if isinstance(kt, int):
                nxt = nxt if nxt < nbt else nxt - nbt
            else:
                nxt = jnp.where(nxt >= nbt, nxt - nbt, nxt)
            pltpu.async_copy(e4.at[nxt * NS + s, c], ebuf[(e + 2) % 4], esem[(e + 2) % 4])
            pltpu.make_async_copy(e4.at[kt * NS + s, c], ebuf[e], esem[e]).wait()
            for j in range(4):
                pltpu.async_copy(ones_v, acc.at[ebuf[e].at[j]], ssem[p], add=True)

        pltpu.async_copy(e4.at[s, c], eb0, es0)
        pltpu.async_copy(e4.at[NS + s, c], eb1, es1)
        emit(0, 0, 0, False)
        emit(1, 1, 1, False)
        emit(2, 2, 0, True)
        emit(3, 3, 1, True)

        def outer(o4, carry):
            for j in range(4):
                emit(o4 * 4 + j, j, j % 2, True)
            return carry

        lax.fori_loop(1, nbt // 4, outer, 0)
        pltpu.make_async_copy(e4.at[s, c], eb0, es0).wait()
        pltpu.make_async_copy(e4.at[NS + s, c], eb1, es1).wait()
        for j in range(4):
            pltpu.make_async_copy(ones_v, acc.at[ebuf[2].at[j]], ssem[0]).wait()
        for j in range(4):
            pltpu.make_async_copy(ones_v, acc.at[ebuf[3].at[j]], ssem[1]).wait()
        plsc.subcore_barrier()

        @pl.when(c == 0)
        def _():
            pltpu.sync_copy(acc.at[pl.ds(s * rpt, rpt)], do_.at[pl.ds(s * rpt, rpt)])

        @pl.when(c == 1)
        def _():
            pltpu.sync_copy(acc.at[pl.ds(s * rpt, rpt)], dn_.at[pl.ds(s * rpt, rpt)])

    return pl.kernel(
        body,
        out_type=(
            jax.ShapeDtypeStruct((np_,), jnp.int32),
            jax.ShapeDtypeStruct((np_,), jnp.int32),
        ),
        mesh=_MESH,
        compiler_params=_SC_PARAMS,
        scratch_types=[
            pltpu.VMEM_SHARED((np_,), jnp.int32),
            pltpu.VMEM((4, 128), jnp.int32),
            pltpu.VMEM((4, 128), jnp.int32),
            pltpu.VMEM((4, 128), jnp.int32),
            pltpu.VMEM((4, 128), jnp.int32),
            pltpu.VMEM((128,), jnp.int32),
            pltpu.SemaphoreType.DMA,
            pltpu.SemaphoreType.DMA,
            pltpu.SemaphoreType.DMA,
            pltpu.SemaphoreType.DMA,
            pltpu.SemaphoreType.DMA,
            pltpu.SemaphoreType.DMA,
        ],
    )


_deg_intra = _make_deg(NP_INTRA, NBT_INTRA)
_deg_sep = _make_deg(NP_SEP, NBT_SEP)


def _make_ep(np_, nbt):
    rpt = np_ // NS

    def body(e4, table, init, av, out, acc,
             eb0, eb1, eb2, eb3, r0, r1, fbuf, abuf,
             es0, es1, es2, es3, gs0, gs1, ss0, ss1):
        c = lax.axis_index("c")
        s = lax.axis_index("s")
        ebuf = (eb0, eb1, eb2, eb3)
        rowsb = (r0, r1)
        esem = (es0, es1, es2, es3)
        gsem = (gs0, gs1)
        ssem = (ss0, ss1)
        tm = table.at[c]
        pltpu.sync_copy(init.at[c, pl.ds(s * rpt, rpt)], acc.at[pl.ds(s * rpt, rpt)])
        pltpu.sync_copy(av.at[pl.ds(s * rpt, rpt)], abuf)
        plsc.subcore_barrier()

        def gathers(e, p):
            for j in range(4):
                pltpu.async_copy(tm.at[ebuf[e].at[0, j]], rowsb[p].at[j], gsem[p])

        def scatters_prev(e, p):
            q = 1 - p
            epv = (e + 3) % 4
            for j in range(4):
                pltpu.make_async_copy(tm.at[ebuf[epv].at[0, j]], rowsb[q].at[j], gsem[q]).wait()
            for j in range(4):
                pltpu.async_copy(rowsb[q].at[j], acc.at[ebuf[epv].at[1, j]], ssem[q], add=True)

        def emit(kt, e, p, first, drain):
            if drain:
                epv = (e + 2) % 4
                for j in range(4):
                    pltpu.make_async_copy(rowsb[p].at[j], acc.at[ebuf[epv].at[1, j]], ssem[p]).wait()
            nxt = kt + 2
            if isinstance(kt, int):
                nxt = nxt if nxt < nbt else nxt - nbt
            else:
                nxt = jnp.where(nxt >= nbt, nxt - nbt, nxt)
            pltpu.async_copy(e4.at[nxt * NS + s], ebuf[(e + 2) % 4], esem[(e + 2) % 4])
            pltpu.make_async_copy(e4.at[kt * NS + s], ebuf[e], esem[e]).wait()
            gathers(e, p)
            if not first:
                scatters_prev(e, p)

        pltpu.async_copy(e4.at[s], eb0, es0)
        pltpu.async_copy(e4.at[NS + s], eb1, es1)
        emit(0, 0, 0, True, False)
        emit(1, 1, 1, False, False)
        emit(2, 2, 0, False, True)
        emit(3, 3, 1, False, True)

        def outer(o4, carry):
            for j in range(4):
                emit(o4 * 4 + j, j, j % 2, False, True)
            return carry

        lax.fori_loop(1, nbt // 4, outer, 0)
        for j in range(4):
            pltpu.make_async_copy(tm.at[ebuf[3].at[0, j]], rowsb[1].at[j], gsem[1]).wait()
        for j in range(4):
            pltpu.async_copy(rowsb[1].at[j], acc.at[ebuf[3].at[1, j]], ssem[1], add=True)
        for j in range(4):
            pltpu.make_async_copy(rowsb[0].at[j], acc.at[ebuf[2].at[1, j]], ssem[0]).wait()
        for j in range(4):
            pltpu.make_async_copy(rowsb[1].at[j], acc.at[ebuf[3].at[1, j]], ssem[1]).wait()
        pltpu.make_async_copy(e4.at[s], eb0, es0).wait()
        pltpu.make_async_copy(e4.at[NS + s], eb1, es1).wait()
        plsc.subcore_barrier()

        def scale_chunk(t, carry):
            r0_ = s * rpt + t * 128
            pltpu.sync_copy(acc.at[pl.ds(r0_, 128)], fbuf)
            base = t * 128 + jnp.zeros((16,), jnp.int32)
            for r in range(128):
                arow = plsc.load_gather(abuf, [base + r])
                fbuf[r] = fbuf[r] * arow
            pltpu.sync_copy(fbuf, out.at[c, pl.ds(r0_, 128)])
            return carry

        lax.fori_loop(0, rpt // 128, scale_chunk, 0)

    return pl.kernel(
        body,
        out_type=jax.ShapeDtypeStruct((NC, np_, 16), jnp.float32),
        mesh=_MESH,
        compiler_params=_SC_PARAMS,
        scratch_types=[
            pltpu.VMEM_SHARED((np_, 16), jnp.float32),
            pltpu.VMEM((2, 4, 128), jnp.int32),
            pltpu.VMEM((2, 4, 128), jnp.int32),
            pltpu.VMEM((2, 4, 128), jnp.int32),
            pltpu.VMEM((2, 4, 128), jnp.int32),
            pltpu.VMEM((4, 128, 16), jnp.float32),
            pltpu.VMEM((4, 128, 16), jnp.float32),
            pltpu.VMEM((128, 16), jnp.float32),
            pltpu.VMEM((rpt,), jnp.float32),
            pltpu.SemaphoreType.DMA,
            pltpu.SemaphoreType.DMA,
            pltpu.SemaphoreType.DMA,
            pltpu.SemaphoreType.DMA,
            pltpu.SemaphoreType.DMA,
            pltpu.SemaphoreType.DMA,
            pltpu.SemaphoreType.DMA,
            pltpu.SemaphoreType.DMA,
        ],
    )


_ep_intra = _make_ep(NP_INTRA, NBT_INTRA)
_ep_sep = _make_ep(NP_SEP, NBT_SEP)


def _fin_body(u2d, is2d, ii2d, h0, h1, hi, uout, iout, idxb, rows, sem):
    c = lax.axis_index("c")
    s = lax.axis_index("s")
    tabs = (h0.at[c], h1.at[c], hi.at[c])
    for q in range(2):
        grp = s * 2 + q
        pltpu.sync_copy(u2d.at[grp], idxb)
        for t in range(3):
            pltpu.async_copy(tabs[t].at[idxb], rows, sem).wait()
            pltpu.sync_copy(rows, uout.at[c, t, pl.ds(grp * 128, 128)])
        pltpu.sync_copy(is2d.at[grp], idxb)
        for t in range(2):
            pltpu.async_copy(tabs[t].at[idxb], rows, sem).wait()
            pltpu.sync_copy(rows, iout.at[c, t, pl.ds(grp * 128, 128)])
        pltpu.sync_copy(ii2d.at[grp], idxb)
        pltpu.async_copy(tabs[2].at[idxb], rows, sem).wait()
        pltpu.sync_copy(rows, iout.at[c, 2, pl.ds(grp * 128, 128)])


_fin_gather = functools.partial(
    pl.kernel,
    _fin_body,
    out_type=(
        jax.ShapeDtypeStruct((NC, 3, B, 16), jnp.float32),
        jax.ShapeDtypeStruct((NC, 3, B, 16), jnp.float32),
    ),
    mesh=_MESH,
    compiler_params=_SC_PARAMS,
    scratch_types=[
        pltpu.VMEM((128,), jnp.int32),
        pltpu.VMEM((128, 16), jnp.float32),
        pltpu.SemaphoreType.DMA,
    ],
)()


_BN = 2000


def _prep_tc_body(do_ref, dn_ref, x_ref, g1_ref, init_ref, a1_ref, a2_ref):
    isdo = lax.rsqrt(jnp.maximum(do_ref[...], 1).astype(jnp.float32))
    isdi = lax.rsqrt(jnp.maximum(dn_ref[...], 1).astype(jnp.float32))
    x = x_ref[...]
    g1_ref[0] = x[:, :16] * isdo
    g1_ref[1] = x[:, 16:] * isdo
    w = (1.0 - ALPHA) * isdi
    iscale = ALPHA / w
    init_ref[0] = x[:, :16] * iscale
    init_ref[1] = x[:, 16:] * iscale
    a1_ref[...] = w * isdo
    a2_ref[...] = w


def _prep_tc(deg_out, deg_in, x, np_):
    nb = x.shape[0] // _BN
    row = pl.BlockSpec((_BN, 1), lambda i: (i, 0))
    tab = pl.BlockSpec((NC, _BN, 16), lambda i: (0, i, 0))
    return pl.pallas_call(
        _prep_tc_body,
        grid=(nb,),
        in_specs=[row, row, pl.BlockSpec((_BN, HALF), lambda i: (i, 0))],
        out_specs=[tab, tab, row, row],
        out_shape=[
            jax.ShapeDtypeStruct((NC, np_, 16), jnp.float32),
            jax.ShapeDtypeStruct((NC, np_, 16), jnp.float32),
            jax.ShapeDtypeStruct((np_, 1), jnp.float32),
            jax.ShapeDtypeStruct((np_, 1), jnp.float32),
        ],
    )(deg_out.reshape(np_, 1), deg_in.reshape(np_, 1), x)


_BF = 1000


def _dot_tc_body(u_ref, i_ref, g_ref):
    u = u_ref[...]
    i = i_ref[...]
    g = jnp.zeros((B,), jnp.float32)
    for c in range(NC):
        u_int = 0.5 * (u[c, 0] + u[c, 1])
        i_int = 0.5 * (i[c, 0] + i[c, 1])
        g = g + jnp.sum(u_int * i_int + u[c, 2] * i[c, 2], axis=1)
    g_ref[...] = g


def _dot_tc(u_rows, i_rows):
    return pl.pallas_call(
        _dot_tc_body,
        out_shape=jax.ShapeDtypeStruct((B,), jnp.float32),
    )(u_rows, i_rows)


def kernel(users, items, edge_index_intra, edge_index_sep0, edge_index_sep1,
           emb_user_d0, emb_item_d0, aggr_user, aggr_item):
    x_intra = jnp.concatenate([emb_user_d0, emb_item_d0], axis=0)
    x_sep = jnp.concatenate([aggr_user, aggr_item], axis=0)

    e4_i = _pad_edges(edge_index_intra[0], edge_index_intra[1], N_INTRA, E_INTRA_P)
    e4_s0 = _pad_edges(edge_index_sep0[0], edge_index_sep0[1], N_SEP, E_SEP_P)
    e4_s1 = _pad_edges(edge_index_sep1[0], edge_index_sep1[1], N_SEP, E_SEP_P)

    zi = jnp.zeros((NP_SEP,), jnp.int32)

    do_i, dn_i = _deg_intra(e4_i, zi)
    do_s0, dn_s0 = _deg_sep(e4_s0, zi)
    do_s1, dn_s1 = _deg_sep(e4_s1, zi)

    g1_i, init_i, a1_i, a2_i = _prep_tc(do_i, dn_i, x_intra, NP_INTRA)
    g1_s0, init_s0, a1_s0, a2_s0 = _prep_tc(do_s0, dn_s0, x_sep, NP_SEP)
    g1_s1, init_s1, a1_s1, a2_s1 = _prep_tc(do_s1, dn_s1, x_sep, NP_SEP)

    g2_i = _ep_intra(e4_i, g1_i, init_i, a1_i.reshape(NP_INTRA))
    g2_s0 = _ep_sep(e4_s0, g1_s0, init_s0, a1_s0.reshape(NP_SEP))
    g2_s1 = _ep_sep(e4_s1, g1_s1, init_s1, a1_s1.reshape(NP_SEP))

    h2_i = _ep_intra(e4_i, g2_i, init_i, a2_i.reshape(NP_INTRA))
    h2_s0 = _ep_sep(e4_s0, g2_s0, init_s0, a2_s0.reshape(NP_SEP))
    h2_s1 = _ep_sep(e4_s1, g2_s1, init_s1, a2_s1.reshape(NP_SEP))

    u2d = users.astype(jnp.int32).reshape(B // 128, 128)
    is2d = (items.astype(jnp.int32) + 50000).reshape(B // 128, 128)
    ii2d = (items.astype(jnp.int32) + N_U0).reshape(B // 128, 128)
    u_rows, i_rows = _fin_gather(u2d, is2d, ii2d, h2_s0, h2_s1, h2_i)
    return _dot_tc(u_rows, i_rows)


# speedup vs baseline: 1.0810x; 1.0006x over previous
import functools

import jax
import jax.numpy as jnp
from jax import lax
from jax.experimental import pallas as pl
from jax.experimental.pallas import tpu as pltpu
from jax.experimental.pallas import tpu_sc as plsc

ALPHA = 0.1
HALF = 32
N_U0 = 25000
N_I0 = 25000
N_INTRA = 50000
N_SEP = 100000
B = 4096

NC = 2
NS = 16

NP_INTRA = 51200
NP_SEP = 100352
NBT_INTRA = 104
NBT_SEP = 200
E_INTRA_P = NBT_INTRA * NS * 512
E_SEP_P = NBT_SEP * NS * 512

_MESH = plsc.VectorSubcoreMesh(
    core_axis_name="c", subcore_axis_name="s", num_cores=NC, num_subcores=NS
)
_SC_PARAMS = pltpu.CompilerParams(
    use_tc_tiling_on_sc=False, needs_layout_passes=False
)


def _pad_edges(src, dst, n, e_pad):
    e = src.shape[0]
    fill = n + (jnp.arange(e_pad - e, dtype=jnp.int32) % 128)
    s1 = jnp.concatenate([src.astype(jnp.int32), fill]).reshape(-1, 4, 128)
    d1 = jnp.concatenate([dst.astype(jnp.int32), fill]).reshape(-1, 4, 128)
    return jnp.stack([s1, d1], axis=1)


def _make_deg(np_, nbt):
    rpt = np_ // NS

    def body(e4, zi, do_, dn_, acc,
             eb0, eb1, eb2, eb3, ones_v,
             es0, es1, es2, es3, ss0, ss1):
        c = lax.axis_index("c")
        s = lax.axis_index("s")
        ebuf = (eb0, eb1, eb2, eb3)
        esem = (es0, es1, es2, es3)
        ssem = (ss0, ss1)
        for j in range(8):
            ones_v[pl.ds(j * 16, 16)] = jnp.ones((16,), jnp.int32)
        pltpu.sync_copy(zi.at[pl.ds(0, rpt)], acc.at[pl.ds(s * rpt, rpt)])
        plsc.subcore_barrier()

        def emit(kt, e, p, drain):
            if drain:
                epv = (e + 2) % 4
                for j in range(4):
                    pltpu.make_async_copy(ones_v, acc.at[ebuf[epv].at[j]], ssem[p]).wait()
            nxt = kt + 2
            if isinstance(kt, int):
                nxt = nxt if nxt < nbt else nxt - nbt
            else:
                nxt = jnp.where(nxt >= nbt, nxt - nbt, nxt)
            pltpu.async_copy(e4.at[nxt * NS + s, c], ebuf[(e + 2) % 4], esem[(e + 2) % 4])
            pltpu.make_async_copy(e4.at[kt * NS + s, c], ebuf[e], esem[e]).wait()
            for j in range(4):
                pltpu.async_copy(ones_v, acc.at[ebuf[e].at[j]], ssem[p], add=True)

        pltpu.async_copy(e4.at[s, c], eb0, es0)
        pltpu.async_copy(e4.at[NS + s, c], eb1, es1)
        emit(0, 0, 0, False)
        emit(1, 1, 1, False)
        emit(2, 2, 0, True)
        emit(3, 3, 1, True)

        def outer(o4, carry):
            for j in range(4):
                emit(o4 * 4 + j, j, j % 2, True)
            return carry

        lax.fori_loop(1, nbt // 4, outer, 0)
        pltpu.make_async_copy(e4.at[s, c], eb0, es0).wait()
        pltpu.make_async_copy(e4.at[NS + s, c], eb1, es1).wait()
        for j in range(4):
            pltpu.make_async_copy(ones_v, acc.at[ebuf[2].at[j]], ssem[0]).wait()
        for j in range(4):
            pltpu.make_async_copy(ones_v, acc.at[ebuf[3].at[j]], ssem[1]).wait()
        plsc.subcore_barrier()

        @pl.when(c == 0)
        def _():
            pltpu.sync_copy(acc.at[pl.ds(s * rpt, rpt)], do_.at[pl.ds(s * rpt, rpt)])

        @pl.when(c == 1)
        def _():
            pltpu.sync_copy(acc.at[pl.ds(s * rpt, rpt)], dn_.at[pl.ds(s * rpt, rpt)])

    return pl.kernel(
        body,
        out_type=(
            jax.ShapeDtypeStruct((np_,), jnp.int32),
            jax.ShapeDtypeStruct((np_,), jnp.int32),
        ),
        mesh=_MESH,
        compiler_params=_SC_PARAMS,
        scratch_types=[
            pltpu.VMEM_SHARED((np_,), jnp.int32),
            pltpu.VMEM((4, 128), jnp.int32),
            pltpu.VMEM((4, 128), jnp.int32),
            pltpu.VMEM((4, 128), jnp.int32),
            pltpu.VMEM((4, 128), jnp.int32),
            pltpu.VMEM((128,), jnp.int32),
            pltpu.SemaphoreType.DMA,
            pltpu.SemaphoreType.DMA,
            pltpu.SemaphoreType.DMA,
            pltpu.SemaphoreType.DMA,
            pltpu.SemaphoreType.DMA,
            pltpu.SemaphoreType.DMA,
        ],
    )


_deg_intra = _make_deg(NP_INTRA, NBT_INTRA)
_deg_sep = _make_deg(NP_SEP, NBT_SEP)


def _make_ep(np_, nbt):
    rpt = np_ // NS

    def body(e4, table, init, av, out, acc,
             eb0, eb1, eb2, eb3, r0, r1, abuf,
             es0, es1, es2, es3, gs0, gs1, ss0, ss1):
        c = lax.axis_index("c")
        s = lax.axis_index("s")
        ebuf = (eb0, eb1, eb2, eb3)
        rowsb = (r0, r1)
        esem = (es0, es1, es2, es3)
        gsem = (gs0, gs1)
        ssem = (ss0, ss1)
        tm = table.at[c]
        pltpu.sync_copy(init.at[c, pl.ds(s * rpt, rpt)], acc.at[pl.ds(s * rpt, rpt)])
        pltpu.sync_copy(av.at[pl.ds(s * rpt, rpt)], abuf)
        plsc.subcore_barrier()

        def gathers(e, p):
            for j in range(4):
                pltpu.async_copy(tm.at[ebuf[e].at[0, j]], rowsb[p].at[j], gsem[p])

        def scatters_prev(e, p):
            q = 1 - p
            epv = (e + 3) % 4
            for j in range(4):
                pltpu.make_async_copy(tm.at[ebuf[epv].at[0, j]], rowsb[q].at[j], gsem[q]).wait()
            for j in range(4):
                pltpu.async_copy(rowsb[q].at[j], acc.at[ebuf[epv].at[1, j]], ssem[q], add=True)

        def emit(kt, e, p, first, drain):
            if drain:
                epv = (e + 2) % 4
                for j in range(4):
                    pltpu.make_async_copy(rowsb[p].at[j], acc.at[ebuf[epv].at[1, j]], ssem[p]).wait()
            nxt = kt + 2
            if isinstance(kt, int):
                nxt = nxt if nxt < nbt else nxt - nbt
            else:
                nxt = jnp.where(nxt >= nbt, nxt - nbt, nxt)
            pltpu.async_copy(e4.at[nxt * NS + s], ebuf[(e + 2) % 4], esem[(e + 2) % 4])
            pltpu.make_async_copy(e4.at[kt * NS + s], ebuf[e], esem[e]).wait()
            gathers(e, p)
            if not first:
                scatters_prev(e, p)

        pltpu.async_copy(e4.at[s], eb0, es0)
        pltpu.async_copy(e4.at[NS + s], eb1, es1)
        emit(0, 0, 0, True, False)
        emit(1, 1, 1, False, False)
        emit(2, 2, 0, False, True)
        emit(3, 3, 1, False, True)

        def outer(o4, carry):
            for j in range(4):
                emit(o4 * 4 + j, j, j % 2, False, True)
            return carry

        lax.fori_loop(1, nbt // 4, outer, 0)
        for j in range(4):
            pltpu.make_async_copy(tm.at[ebuf[3].at[0, j]], rowsb[1].at[j], gsem[1]).wait()
        for j in range(4):
            pltpu.async_copy(rowsb[1].at[j], acc.at[ebuf[3].at[1, j]], ssem[1], add=True)
        for j in range(4):
            pltpu.make_async_copy(rowsb[0].at[j], acc.at[ebuf[2].at[1, j]], ssem[0]).wait()
        for j in range(4):
            pltpu.make_async_copy(rowsb[1].at[j], acc.at[ebuf[3].at[1, j]], ssem[1]).wait()
        pltpu.make_async_copy(e4.at[s], eb0, es0).wait()
        pltpu.make_async_copy(e4.at[NS + s], eb1, es1).wait()
        plsc.subcore_barrier()

        def scale_chunk(t, carry):
            r0_ = s * rpt + t * 128
            pltpu.sync_copy(acc.at[pl.ds(r0_, 128)], r0.at[0])
            base = t * 128 + jnp.zeros((16,), jnp.int32)
            for r in range(128):
                arow = plsc.load_gather(abuf, [base + r])
                r0[0, r] = r0[0, r] * arow
            pltpu.sync_copy(r0.at[0], out.at[c, pl.ds(r0_, 128)])
            return carry

        lax.fori_loop(0, rpt // 128, scale_chunk, 0)

    return pl.kernel(
        body,
        out_type=jax.ShapeDtypeStruct((NC, np_, 16), jnp.float32),
        mesh=_MESH,
        compiler_params=_SC_PARAMS,
        scratch_types=[
            pltpu.VMEM_SHARED((np_, 16), jnp.float32),
            pltpu.VMEM((2, 4, 128), jnp.int32),
            pltpu.VMEM((2, 4, 128), jnp.int32),
            pltpu.VMEM((2, 4, 128), jnp.int32),
            pltpu.VMEM((2, 4, 128), jnp.int32),
            pltpu.VMEM((4, 128, 16), jnp.float32),
            pltpu.VMEM((4, 128, 16), jnp.float32),
            pltpu.VMEM((rpt,), jnp.float32),
            pltpu.SemaphoreType.DMA,
            pltpu.SemaphoreType.DMA,
            pltpu.SemaphoreType.DMA,
            pltpu.SemaphoreType.DMA,
            pltpu.SemaphoreType.DMA,
            pltpu.SemaphoreType.DMA,
            pltpu.SemaphoreType.DMA,
            pltpu.SemaphoreType.DMA,
        ],
    )


_ep_intra = _make_ep(NP_INTRA, NBT_INTRA)
_ep_sep = _make_ep(NP_SEP, NBT_SEP)


def _fin_body(u2d, is2d, ii2d, h0, h1, hi, uout, iout, idxb, rows, sem):
    c = lax.axis_index("c")
    s = lax.axis_index("s")
    tabs = (h0.at[c], h1.at[c], hi.at[c])
    for q in range(2):
        grp = s * 2 + q
        pltpu.sync_copy(u2d.at[grp], idxb)
        for t in range(3):
            pltpu.async_copy(tabs[t].at[idxb], rows, sem).wait()
            pltpu.sync_copy(rows, uout.at[c, t, pl.ds(grp * 128, 128)])
        pltpu.sync_copy(is2d.at[grp], idxb)
        for t in range(2):
            pltpu.async_copy(tabs[t].at[idxb], rows, sem).wait()
            pltpu.sync_copy(rows, iout.at[c, t, pl.ds(grp * 128, 128)])
        pltpu.sync_copy(ii2d.at[grp], idxb)
        pltpu.async_copy(tabs[2].at[idxb], rows, sem).wait()
        pltpu.sync_copy(rows, iout.at[c, 2, pl.ds(grp * 128, 128)])


_fin_gather = functools.partial(
    pl.kernel,
    _fin_body,
    out_type=(
        jax.ShapeDtypeStruct((NC, 3, B, 16), jnp.float32),
        jax.ShapeDtypeStruct((NC, 3, B, 16), jnp.float32),
    ),
    mesh=_MESH,
    compiler_params=_SC_PARAMS,
    scratch_types=[
        pltpu.VMEM((128,), jnp.int32),
        pltpu.VMEM((128, 16), jnp.float32),
        pltpu.SemaphoreType.DMA,
    ],
)()


_BN = 2000


def _prep_tc_body(do_ref, dn_ref, x_ref, g1_ref, init_ref, a1_ref, a2_ref):
    isdo = lax.rsqrt(jnp.maximum(do_ref[...], 1).astype(jnp.float32))
    isdi = lax.rsqrt(jnp.maximum(dn_ref[...], 1).astype(jnp.float32))
    x = x_ref[...]
    g1_ref[0] = x[:, :16] * isdo
    g1_ref[1] = x[:, 16:] * isdo
    w = (1.0 - ALPHA) * isdi
    iscale = ALPHA / w
    init_ref[0] = x[:, :16] * iscale
    init_ref[1] = x[:, 16:] * iscale
    a1_ref[...] = w * isdo
    a2_ref[...] = w


def _prep_tc(deg_out, deg_in, x, np_):
    nb = x.shape[0] // _BN
    row = pl.BlockSpec((_BN, 1), lambda i: (i, 0))
    tab = pl.BlockSpec((NC, _BN, 16), lambda i: (0, i, 0))
    return pl.pallas_call(
        _prep_tc_body,
        grid=(nb,),
        in_specs=[row, row, pl.BlockSpec((_BN, HALF), lambda i: (i, 0))],
        out_specs=[tab, tab, row, row],
        out_shape=[
            jax.ShapeDtypeStruct((NC, np_, 16), jnp.float32),
            jax.ShapeDtypeStruct((NC, np_, 16), jnp.float32),
            jax.ShapeDtypeStruct((np_, 1), jnp.float32),
            jax.ShapeDtypeStruct((np_, 1), jnp.float32),
        ],
    )(deg_out.reshape(np_, 1), deg_in.reshape(np_, 1), x)


_BF = 1000


def _dot_tc_body(u_ref, i_ref, g_ref):
    u = u_ref[...]
    i = i_ref[...]
    g = jnp.zeros((B,), jnp.float32)
    for c in range(NC):
        u_int = 0.5 * (u[c, 0] + u[c, 1])
        i_int = 0.5 * (i[c, 0] + i[c, 1])
        g = g + jnp.sum(u_int * i_int + u[c, 2] * i[c, 2], axis=1)
    g_ref[...] = g


def _dot_tc(u_rows, i_rows):
    return pl.pallas_call(
        _dot_tc_body,
        out_shape=jax.ShapeDtypeStruct((B,), jnp.float32),
    )(u_rows, i_rows)


def kernel(users, items, edge_index_intra, edge_index_sep0, edge_index_sep1,
           emb_user_d0, emb_item_d0, aggr_user, aggr_item):
    x_intra = jnp.concatenate([emb_user_d0, emb_item_d0], axis=0)
    x_sep = jnp.concatenate([aggr_user, aggr_item], axis=0)

    e4_i = _pad_edges(edge_index_intra[0], edge_index_intra[1], N_INTRA, E_INTRA_P)
    e4_s0 = _pad_edges(edge_index_sep0[0], edge_index_sep0[1], N_SEP, E_SEP_P)
    e4_s1 = _pad_edges(edge_index_sep1[0], edge_index_sep1[1], N_SEP, E_SEP_P)

    zi = jnp.zeros((NP_SEP,), jnp.int32)

    do_i, dn_i = _deg_intra(e4_i, zi)
    do_s0, dn_s0 = _deg_sep(e4_s0, zi)
    do_s1, dn_s1 = _deg_sep(e4_s1, zi)

    g1_i, init_i, a1_i, a2_i = _prep_tc(do_i, dn_i, x_intra, NP_INTRA)
    g1_s0, init_s0, a1_s0, a2_s0 = _prep_tc(do_s0, dn_s0, x_sep, NP_SEP)
    g1_s1, init_s1, a1_s1, a2_s1 = _prep_tc(do_s1, dn_s1, x_sep, NP_SEP)

    g2_i = _ep_intra(e4_i, g1_i, init_i, a1_i.reshape(NP_INTRA))
    g2_s0 = _ep_sep(e4_s0, g1_s0, init_s0, a1_s0.reshape(NP_SEP))
    g2_s1 = _ep_sep(e4_s1, g1_s1, init_s1, a1_s1.reshape(NP_SEP))

    h2_i = _ep_intra(e4_i, g2_i, init_i, a2_i.reshape(NP_INTRA))
    h2_s0 = _ep_sep(e4_s0, g2_s0, init_s0, a2_s0.reshape(NP_SEP))
    h2_s1 = _ep_sep(e4_s1, g2_s1, init_s1, a2_s1.reshape(NP_SEP))

    u2d = users.astype(jnp.int32).reshape(B // 128, 128)
    is2d = (items.astype(jnp.int32) + 50000).reshape(B // 128, 128)
    ii2d = (items.astype(jnp.int32) + N_U0).reshape(B // 128, 128)
    u_rows, i_rows = _fin_gather(u2d, is2d, ii2d, h2_s0, h2_s1, h2_i)
    return _dot_tc(u_rows, i_rows)
